# baseline (speedup 1.0000x reference)
import functools

import jax
import jax.numpy as jnp
from jax import lax
from jax.experimental import pallas as pl
from jax.experimental.pallas import tpu as pltpu
from jax.experimental.pallas import tpu_sc as plsc

_OPF = 140
_EMB = 128
_CFG = 18
_OUT = 128

_NC = 2
_NS = 16
_NW = _NC * _NS
_CH = 200


def _prep_kernel(emb_ref, wemb_ref, wcfg_ref, cfgwt_ref, opw_ref, b_ref,
                 proj_ref, wcfgf_ref):
    emb = emb_ref[...]
    norm = jnp.sqrt(jnp.sum(emb * emb, axis=1, keepdims=True))
    scale = jnp.minimum(1.0, 1.0 / jnp.maximum(norm, 1e-7)) * opw_ref[0, 0]
    proj_ref[...] = (
        jnp.dot(emb * scale, wemb_ref[...], preferred_element_type=jnp.float32)
        + b_ref[...]
    )
    wcfgf_ref[...] = wcfg_ref[...] * cfgwt_ref[...]


def _main_kernel(opf_ref, cfg_ref, g_ref, wop_ref, wcfgf_ref, out_ref):
    acc = jnp.dot(opf_ref[...], wop_ref[...],
                  preferred_element_type=jnp.float32)
    acc += jnp.dot(cfg_ref[...], wcfgf_ref[...],
                   preferred_element_type=jnp.float32)
    out_ref[...] = acc + g_ref[...]


def _make_sc_gather(n):
    n_chunks = n // _CH
    assert n_chunks * _CH == n and _CH % 8 == 0

    mesh = plsc.VectorSubcoreMesh(core_axis_name="c", subcore_axis_name="s")

    @functools.partial(
        pl.kernel, mesh=mesh,
        out_type=jax.ShapeDtypeStruct((n, _OUT), jnp.float32),
        scratch_types=[
            pltpu.VMEM((_CH,), jnp.int32),
            pltpu.VMEM((_CH, _OUT), jnp.float32),
            pltpu.SemaphoreType.DMA,
        ],
    )
    def gather_k(proj_hbm, idx_hbm, g_hbm, idx_v, rows_v, sem):
        wid = lax.axis_index("s") * _NC + lax.axis_index("c")
        n_iters = (n_chunks - wid + _NW - 1) // _NW

        def body(k, carry):
            base = (wid + k * _NW) * _CH
            pltpu.sync_copy(idx_hbm.at[pl.ds(base, _CH)], idx_v)
            pltpu.async_copy(proj_hbm.at[idx_v], rows_v, sem).wait()
            pltpu.sync_copy(rows_v, g_hbm.at[pl.ds(base, _CH), :])
            return carry

        lax.fori_loop(0, n_iters, body, 0)

    return gather_k


def kernel(op_feats, config_feats, emb_table, op_weights, config_weights,
           W, b, op_code):
    n = op_feats.shape[0]
    w_op = W[0:_OPF]
    w_emb = W[_OPF:_OPF + _EMB]
    w_cfg = W[_OPF + _EMB:]
    cfgw_t = config_weights.reshape(_CFG, 1)
    b2 = b.reshape(1, _OUT)

    emb_proj, w_cfg_f = pl.pallas_call(
        _prep_kernel,
        out_shape=[
            jax.ShapeDtypeStruct((_EMB, _OUT), jnp.float32),
            jax.ShapeDtypeStruct((_CFG, _OUT), jnp.float32),
        ],
    )(emb_table, w_emb, w_cfg, cfgw_t, op_weights, b2)

    idx = op_code.astype(jnp.int32)
    g = _make_sc_gather(n)(emb_proj, idx)

    blk = 10000
    grid = n // blk
    assert grid * blk == n

    out = pl.pallas_call(
        _main_kernel,
        grid=(grid,),
        in_specs=[
            pl.BlockSpec((blk, _OPF), lambda i: (i, 0)),
            pl.BlockSpec((blk, _CFG), lambda i: (i, 0)),
            pl.BlockSpec((blk, _OUT), lambda i: (i, 0)),
            pl.BlockSpec((_OPF, _OUT), lambda i: (0, 0)),
            pl.BlockSpec((_CFG, _OUT), lambda i: (0, 0)),
        ],
        out_specs=pl.BlockSpec((blk, _OUT), lambda i: (i, 0)),
        out_shape=jax.ShapeDtypeStruct((n, _OUT), jnp.float32),
        compiler_params=pltpu.CompilerParams(
            dimension_semantics=("parallel",)),
    )(op_feats, config_feats, g, w_op, w_cfg_f)
    return out


# speedup vs baseline: 1.7143x; 1.7143x over previous
import jax
import jax.numpy as jnp
from jax.experimental import pallas as pl
from jax.experimental.pallas import tpu as pltpu

_OPF = 140
_EMB = 128
_CFG = 18
_OUT = 128


def _main_kernel(opf_ref, cfg_ref, idx_ref, wop_ref, wemb_ref, wcfg_ref,
                 emb_ref, cfgw_ref, opw_ref, b_ref, out_ref, proj_scr):
    blk = opf_ref.shape[0]

    @pl.when(pl.program_id(0) == 0)
    def _prep():
        emb = emb_ref[...]
        norm = jnp.sqrt(jnp.sum(emb * emb, axis=1, keepdims=True))
        scale = jnp.minimum(1.0, 1.0 / jnp.maximum(norm, 1e-7)) * opw_ref[0, 0]
        proj_scr[...] = (
            jnp.dot(emb * scale, wemb_ref[...],
                    preferred_element_type=jnp.float32)
            + b_ref[...]
        )

    idx = idx_ref[...].reshape(1, blk)
    oh_t = (jax.lax.broadcasted_iota(jnp.int32, (_EMB, blk), 0) == idx
            ).astype(jnp.float32)
    acc = jnp.dot(opf_ref[...], wop_ref[...],
                  preferred_element_type=jnp.float32)
    acc += jnp.dot(cfg_ref[...] * cfgw_ref[...], wcfg_ref[...],
                   preferred_element_type=jnp.float32)
    acc += jax.lax.dot_general(oh_t, proj_scr[...],
                               (((0,), (0,)), ((), ())),
                               preferred_element_type=jnp.float32)
    out_ref[...] = acc


def kernel(op_feats, config_feats, emb_table, op_weights, config_weights,
           W, b, op_code):
    n = op_feats.shape[0]
    w_op = W[0:_OPF]
    w_emb = W[_OPF:_OPF + _EMB]
    w_cfg = W[_OPF + _EMB:]
    b2 = b.reshape(1, _OUT)

    blk = 10000
    grid = n // blk
    assert grid * blk == n
    idx = op_code.astype(jnp.int32).reshape(grid, 1, blk)

    out = pl.pallas_call(
        _main_kernel,
        grid=(grid,),
        in_specs=[
            pl.BlockSpec((blk, _OPF), lambda i: (i, 0)),
            pl.BlockSpec((blk, _CFG), lambda i: (i, 0)),
            pl.BlockSpec((1, 1, blk), lambda i: (i, 0, 0)),
            pl.BlockSpec((_OPF, _OUT), lambda i: (0, 0)),
            pl.BlockSpec((_EMB, _OUT), lambda i: (0, 0)),
            pl.BlockSpec((_CFG, _OUT), lambda i: (0, 0)),
            pl.BlockSpec((_EMB, _EMB), lambda i: (0, 0)),
            pl.BlockSpec((1, _CFG), lambda i: (0, 0)),
            pl.BlockSpec((1, 1), lambda i: (0, 0)),
            pl.BlockSpec((1, _OUT), lambda i: (0, 0)),
        ],
        out_specs=pl.BlockSpec((blk, _OUT), lambda i: (i, 0)),
        out_shape=jax.ShapeDtypeStruct((n, _OUT), jnp.float32),
        scratch_shapes=[pltpu.VMEM((_EMB, _OUT), jnp.float32)],
        compiler_params=pltpu.CompilerParams(
            dimension_semantics=("arbitrary",)),
    )(op_feats, config_feats, idx, w_op, w_emb, w_cfg, emb_table,
      config_weights, op_weights, b2)
    return out
